# Initial kernel scaffold; baseline (speedup 1.0000x reference)
import jax
import jax.numpy as jnp
from jax.experimental import pallas as pl


def kernel(xyz1, xyz2, points1, points2, W0, gamma0, beta0, W1, gamma1, beta1):
    raise NotImplementedError("write your pallas kernel here")


# speedup vs baseline: 36.0552x; 36.0552x over previous
import functools

import jax
import jax.numpy as jnp
from jax.experimental import pallas as pl
from jax.experimental.pallas import tpu as pltpu

_F32_EPS = float(jnp.finfo(jnp.float32).eps)
_BN_EPS = 1e-5


def _stage_a_body(x_ref, yt_ref, p1_ref, p2_ref, w0a_ref, w0b_ref,
                  y0_ref, st_ref, acc_ref, *, n_tiles):
    b = pl.program_id(0)
    t = pl.program_id(1)
    nb = pl.num_programs(0)

    x = x_ref[0]
    yt = yt_ref[0]
    s = yt.shape[1]

    xx = jnp.sum(x * x, axis=1, keepdims=True)
    yy = jnp.sum(yt * yt, axis=0, keepdims=True)
    xy = jnp.dot(x, yt, preferred_element_type=jnp.float32)
    d = jnp.maximum(xx + yy - 2.0 * xy, 0.0)

    col = jax.lax.broadcasted_iota(jnp.int32, d.shape, 1)
    ms, idxs = [], []
    for _ in range(3):
        m = jnp.min(d, axis=1, keepdims=True)
        i = jnp.min(jnp.where(d == m, col, s), axis=1, keepdims=True)
        d = jnp.where(col == i, jnp.inf, d)
        ms.append(m)
        idxs.append(i)

    r0 = 1.0 / (jnp.sqrt(ms[0]) + _F32_EPS)
    r1 = 1.0 / (jnp.sqrt(ms[1]) + _F32_EPS)
    r2 = 1.0 / (jnp.sqrt(ms[2]) + _F32_EPS)
    norm = r0 + r1 + r2
    w0 = r0 / norm
    w1 = r1 / norm
    w2 = r2 / norm

    zero = jnp.zeros_like(d)
    oh = jnp.where(col == idxs[0], w0,
                   jnp.where(col == idxs[1], w1,
                             jnp.where(col == idxs[2], w2, zero)))
    interp = jnp.dot(oh, p2_ref[0], preferred_element_type=jnp.float32)

    y0 = (jnp.dot(p1_ref[0], w0a_ref[...], preferred_element_type=jnp.float32)
          + jnp.dot(interp, w0b_ref[...], preferred_element_type=jnp.float32))
    y0_ref[0] = y0

    @pl.when(jnp.logical_and(b == 0, t == 0))
    def _init():
        acc_ref[...] = jnp.zeros_like(acc_ref)

    acc_ref[0:1, :] += jnp.sum(y0, axis=0, keepdims=True)
    acc_ref[1:2, :] += jnp.sum(y0 * y0, axis=0, keepdims=True)

    @pl.when(jnp.logical_and(b == nb - 1, t == n_tiles - 1))
    def _fin():
        st_ref[...] = acc_ref[...]


def _stage_b_body(y0_ref, sc_ref, sh_ref, w1t_ref, y1_ref, st_ref, acc_ref,
                  *, n_tiles):
    t = pl.program_id(0)
    h = jnp.maximum(y0_ref[...] * sc_ref[...] + sh_ref[...], 0.0)
    y1 = jnp.dot(h, w1t_ref[...], preferred_element_type=jnp.float32)
    y1_ref[...] = y1

    @pl.when(t == 0)
    def _init():
        acc_ref[...] = jnp.zeros_like(acc_ref)

    acc_ref[0:1, :] += jnp.sum(y1, axis=0, keepdims=True)
    acc_ref[1:2, :] += jnp.sum(y1 * y1, axis=0, keepdims=True)

    @pl.when(t == n_tiles - 1)
    def _fin():
        st_ref[...] = acc_ref[...]


def _stage_c_body(y1_ref, sc_ref, sh_ref, out_ref):
    out_ref[...] = jnp.maximum(y1_ref[...] * sc_ref[...] + sh_ref[...], 0.0)


def _scale_shift(stats, count, gamma, beta):
    mean = stats[0] / count
    var = stats[1] / count - mean * mean
    scale = gamma / jnp.sqrt(var + _BN_EPS)
    shift = beta - mean * scale
    return scale.reshape(1, -1), shift.reshape(1, -1)


@jax.jit
def kernel(xyz1, xyz2, points1, points2, W0, gamma0, beta0, W1, gamma1, beta1):
    B, N, _ = xyz1.shape
    S = xyz2.shape[1]
    D1 = points1.shape[2]
    D2 = points2.shape[2]
    TILE_N = 256
    n_tiles_a = N // TILE_N

    xyz2t = jnp.transpose(xyz2, (0, 2, 1))
    w0at = W0[:, :D1].T
    w0bt = W0[:, D1:].T
    w1t = W1.T

    y0, stats0 = pl.pallas_call(
        functools.partial(_stage_a_body, n_tiles=n_tiles_a),
        grid=(B, n_tiles_a),
        in_specs=[
            pl.BlockSpec((1, TILE_N, 3), lambda b, t: (b, t, 0)),
            pl.BlockSpec((1, 3, S), lambda b, t: (b, 0, 0)),
            pl.BlockSpec((1, TILE_N, D1), lambda b, t: (b, t, 0)),
            pl.BlockSpec((1, S, D2), lambda b, t: (b, 0, 0)),
            pl.BlockSpec((D1, 128), lambda b, t: (0, 0)),
            pl.BlockSpec((D2, 128), lambda b, t: (0, 0)),
        ],
        out_specs=[
            pl.BlockSpec((1, TILE_N, 128), lambda b, t: (b, t, 0)),
            pl.BlockSpec((8, 128), lambda b, t: (0, 0)),
        ],
        out_shape=[
            jax.ShapeDtypeStruct((B, N, 128), jnp.float32),
            jax.ShapeDtypeStruct((8, 128), jnp.float32),
        ],
        scratch_shapes=[pltpu.VMEM((8, 128), jnp.float32)],
    )(xyz1, xyz2t, points1, points2, w0at, w0bt)

    count = jnp.float32(B * N)
    scale0, shift0 = _scale_shift(stats0, count, gamma0, beta0)

    y0f = y0.reshape(B * N, 128)
    TILE_R = min(2048, B * N)
    n_tiles_b = (B * N) // TILE_R

    y1, stats1 = pl.pallas_call(
        functools.partial(_stage_b_body, n_tiles=n_tiles_b),
        grid=(n_tiles_b,),
        in_specs=[
            pl.BlockSpec((TILE_R, 128), lambda t: (t, 0)),
            pl.BlockSpec((1, 128), lambda t: (0, 0)),
            pl.BlockSpec((1, 128), lambda t: (0, 0)),
            pl.BlockSpec((128, 128), lambda t: (0, 0)),
        ],
        out_specs=[
            pl.BlockSpec((TILE_R, 128), lambda t: (t, 0)),
            pl.BlockSpec((8, 128), lambda t: (0, 0)),
        ],
        out_shape=[
            jax.ShapeDtypeStruct((B * N, 128), jnp.float32),
            jax.ShapeDtypeStruct((8, 128), jnp.float32),
        ],
        scratch_shapes=[pltpu.VMEM((8, 128), jnp.float32)],
    )(y0f, scale0, shift0, w1t)

    scale1, shift1 = _scale_shift(stats1, count, gamma1, beta1)

    out = pl.pallas_call(
        _stage_c_body,
        grid=(n_tiles_b,),
        in_specs=[
            pl.BlockSpec((TILE_R, 128), lambda t: (t, 0)),
            pl.BlockSpec((1, 128), lambda t: (0, 0)),
            pl.BlockSpec((1, 128), lambda t: (0, 0)),
        ],
        out_specs=pl.BlockSpec((TILE_R, 128), lambda t: (t, 0)),
        out_shape=jax.ShapeDtypeStruct((B * N, 128), jnp.float32),
    )(y1, scale1, shift1)

    return out.reshape(B, N, 128)


# speedup vs baseline: 43.1019x; 1.1954x over previous
import functools

import jax
import jax.numpy as jnp
from jax.experimental import pallas as pl
from jax.experimental.pallas import tpu as pltpu

_F32_EPS = float(jnp.finfo(jnp.float32).eps)
_BN_EPS = 1e-5


def _stage_a_body(x_ref, yt_ref, p1_ref, p2_ref, w0a_ref, w0b_ref,
                  y0_ref, st_ref, acc_ref, *, n_tiles):
    b = pl.program_id(0)
    t = pl.program_id(1)
    nb = pl.num_programs(0)

    x = x_ref[0]
    yt = yt_ref[0]
    s = yt.shape[1]

    xx = jnp.sum(x * x, axis=1, keepdims=True)
    yy = 0.25 * jnp.sum(yt * yt, axis=0, keepdims=True)
    xy = jnp.dot(x, yt, preferred_element_type=jnp.float32)
    d = jnp.maximum((xx + yy) + xy, 0.0)

    colf = jax.lax.broadcasted_iota(jnp.int32, d.shape, 1).astype(jnp.float32)
    big = jnp.float32(s)
    ms, eqcols = [], []
    for k in range(3):
        m = jnp.min(d, axis=1, keepdims=True)
        i = jnp.min(jnp.where(d == m, colf, big), axis=1, keepdims=True)
        eqcol = colf == i
        eqcols.append(eqcol)
        ms.append(m)
        if k < 2:
            d = jnp.where(eqcol, jnp.inf, d)

    r0 = 1.0 / (jnp.sqrt(ms[0]) + _F32_EPS)
    r1 = 1.0 / (jnp.sqrt(ms[1]) + _F32_EPS)
    r2 = 1.0 / (jnp.sqrt(ms[2]) + _F32_EPS)
    norm = r0 + r1 + r2
    w0 = r0 / norm
    w1 = r1 / norm
    w2 = r2 / norm

    zero = jnp.zeros_like(d)
    oh = jnp.where(eqcols[0], w0,
                   jnp.where(eqcols[1], w1,
                             jnp.where(eqcols[2], w2, zero)))
    interp = jnp.dot(oh, p2_ref[0], preferred_element_type=jnp.float32)

    y0 = (jnp.dot(p1_ref[0], w0a_ref[...], preferred_element_type=jnp.float32)
          + jnp.dot(interp, w0b_ref[...], preferred_element_type=jnp.float32))
    y0_ref[0] = y0

    @pl.when(jnp.logical_and(b == 0, t == 0))
    def _init():
        acc_ref[...] = jnp.zeros_like(acc_ref)

    acc_ref[0:1, :] += jnp.sum(y0, axis=0, keepdims=True)
    acc_ref[1:2, :] += jnp.sum(y0 * y0, axis=0, keepdims=True)

    @pl.when(jnp.logical_and(b == nb - 1, t == n_tiles - 1))
    def _fin():
        st_ref[...] = acc_ref[...]


def _stage_b_body(y0_ref, sc_ref, sh_ref, w1t_ref, y1_ref, st_ref, acc_ref,
                  *, n_tiles):
    t = pl.program_id(0)
    h = jnp.maximum(y0_ref[...] * sc_ref[...] + sh_ref[...], 0.0)
    y1 = jnp.dot(h, w1t_ref[...], preferred_element_type=jnp.float32)
    y1_ref[...] = y1

    @pl.when(t == 0)
    def _init():
        acc_ref[...] = jnp.zeros_like(acc_ref)

    acc_ref[0:1, :] += jnp.sum(y1, axis=0, keepdims=True)
    acc_ref[1:2, :] += jnp.sum(y1 * y1, axis=0, keepdims=True)

    @pl.when(t == n_tiles - 1)
    def _fin():
        st_ref[...] = acc_ref[...]


def _stage_c_body(y1_ref, sc_ref, sh_ref, out_ref):
    out_ref[...] = jnp.maximum(y1_ref[...] * sc_ref[...] + sh_ref[...], 0.0)


def _scale_shift(stats, count, gamma, beta):
    mean = stats[0] / count
    var = stats[1] / count - mean * mean
    scale = gamma / jnp.sqrt(var + _BN_EPS)
    shift = beta - mean * scale
    return scale.reshape(1, -1), shift.reshape(1, -1)


@jax.jit
def kernel(xyz1, xyz2, points1, points2, W0, gamma0, beta0, W1, gamma1, beta1):
    B, N, _ = xyz1.shape
    S = xyz2.shape[1]
    D1 = points1.shape[2]
    D2 = points2.shape[2]
    TILE_N = 512
    n_tiles_a = N // TILE_N

    xyz2t = -2.0 * jnp.transpose(xyz2, (0, 2, 1))
    w0at = W0[:, :D1].T
    w0bt = W0[:, D1:].T
    w1t = W1.T

    y0, stats0 = pl.pallas_call(
        functools.partial(_stage_a_body, n_tiles=n_tiles_a),
        grid=(B, n_tiles_a),
        in_specs=[
            pl.BlockSpec((1, TILE_N, 3), lambda b, t: (b, t, 0)),
            pl.BlockSpec((1, 3, S), lambda b, t: (b, 0, 0)),
            pl.BlockSpec((1, TILE_N, D1), lambda b, t: (b, t, 0)),
            pl.BlockSpec((1, S, D2), lambda b, t: (b, 0, 0)),
            pl.BlockSpec((D1, 128), lambda b, t: (0, 0)),
            pl.BlockSpec((D2, 128), lambda b, t: (0, 0)),
        ],
        out_specs=[
            pl.BlockSpec((1, TILE_N, 128), lambda b, t: (b, t, 0)),
            pl.BlockSpec((8, 128), lambda b, t: (0, 0)),
        ],
        out_shape=[
            jax.ShapeDtypeStruct((B, N, 128), jnp.float32),
            jax.ShapeDtypeStruct((8, 128), jnp.float32),
        ],
        scratch_shapes=[pltpu.VMEM((8, 128), jnp.float32)],
    )(xyz1, xyz2t, points1, points2, w0at, w0bt)

    count = jnp.float32(B * N)
    scale0, shift0 = _scale_shift(stats0, count, gamma0, beta0)

    y0f = y0.reshape(B * N, 128)
    TILE_R = min(2048, B * N)
    n_tiles_b = (B * N) // TILE_R

    y1, stats1 = pl.pallas_call(
        functools.partial(_stage_b_body, n_tiles=n_tiles_b),
        grid=(n_tiles_b,),
        in_specs=[
            pl.BlockSpec((TILE_R, 128), lambda t: (t, 0)),
            pl.BlockSpec((1, 128), lambda t: (0, 0)),
            pl.BlockSpec((1, 128), lambda t: (0, 0)),
            pl.BlockSpec((128, 128), lambda t: (0, 0)),
        ],
        out_specs=[
            pl.BlockSpec((TILE_R, 128), lambda t: (t, 0)),
            pl.BlockSpec((8, 128), lambda t: (0, 0)),
        ],
        out_shape=[
            jax.ShapeDtypeStruct((B * N, 128), jnp.float32),
            jax.ShapeDtypeStruct((8, 128), jnp.float32),
        ],
        scratch_shapes=[pltpu.VMEM((8, 128), jnp.float32)],
    )(y0f, scale0, shift0, w1t)

    scale1, shift1 = _scale_shift(stats1, count, gamma1, beta1)

    out = pl.pallas_call(
        _stage_c_body,
        grid=(n_tiles_b,),
        in_specs=[
            pl.BlockSpec((TILE_R, 128), lambda t: (t, 0)),
            pl.BlockSpec((1, 128), lambda t: (0, 0)),
            pl.BlockSpec((1, 128), lambda t: (0, 0)),
        ],
        out_specs=pl.BlockSpec((TILE_R, 128), lambda t: (t, 0)),
        out_shape=jax.ShapeDtypeStruct((B * N, 128), jnp.float32),
    )(y1, scale1, shift1)

    return out.reshape(B, N, 128)
